# baseline (speedup 1.0000x reference)
import dataclasses
import functools

import jax
import jax.numpy as jnp
from jax import lax
from jax.experimental import pallas as pl
from jax.experimental.pallas import tpu as pltpu
from jax.experimental.pallas import tpu_sc as plsc

N = 10000
E = 320000
D = 128

NC = 2
NS = 16
NW = NC * NS
EPW = E // NW
C = 100
CH = EPW // C
NP = 10240
RPS = NP // NS

DPS = E // NS
DVR = (DPS + 127) // 128 + (1 if DPS % 128 else 0)
DVR = -(-DPS // 128)
DPAD = DVR * 128


def _mesh():
    return plsc.VectorSubcoreMesh(core_axis_name="c", subcore_axis_name="s",
                                  num_cores=NC, num_subcores=NS)


def _cp():
    cp = pltpu.CompilerParams()
    if "needs_layout_passes" in pltpu.CompilerParams.__dataclass_fields__:
        cp = dataclasses.replace(cp, needs_layout_passes=False)
    return cp


def _sc_degree(dstp):
    @functools.partial(
        pl.kernel,
        out_type=jax.ShapeDtypeStruct((NP,), jnp.float32),
        mesh=_mesh(),
        scratch_types=[
            pltpu.VMEM((DVR, 128), jnp.int32),
            pltpu.VMEM((NP,), jnp.float32),
            pltpu.VMEM((RPS,), jnp.float32),
            pltpu.VMEM((RPS,), jnp.float32),
            pltpu.VMEM_SHARED((NS, NP), jnp.float32),
        ],
        compiler_params=_cp(),
    )
    def k(dst_hbm, out_hbm, idx_v, hist_v, acc_l, tmp_l, sh):
        c = lax.axis_index("c")
        s = lax.axis_index("s")
        pltpu.sync_copy(dst_hbm.at[s], idx_v)

        @pl.loop(0, NP, step=16)
        def _(i):
            hist_v[pl.ds(i, 16)] = jnp.zeros((16,), jnp.float32)

        ones = jnp.ones((16,), jnp.float32)

        @pl.loop(0, DVR)
        def _(i):
            for l in range(8):
                plsc.addupdate_scatter(
                    hist_v, [idx_v[i, pl.ds(l * 16, 16)]], ones)

        pltpu.sync_copy(hist_v, sh.at[s])
        plsc.subcore_barrier()

        pltpu.sync_copy(sh.at[0, pl.ds(s * RPS, RPS)], acc_l)
        for w in range(1, NS):
            pltpu.sync_copy(sh.at[w, pl.ds(s * RPS, RPS)], tmp_l)

            @pl.loop(0, RPS, step=16)
            def _(i):
                acc_l[pl.ds(i, 16)] += tmp_l[pl.ds(i, 16)]

        @pl.when(c == 0)
        def _():
            pltpu.sync_copy(acc_l, out_hbm.at[pl.ds(s * RPS, RPS)])

    return k(dstp)


def _sc_scatter(hn, idx3, zeros128):
    NB = 2

    @functools.partial(
        pl.kernel,
        out_type=jax.ShapeDtypeStruct((NC, NP, D), jnp.float32),
        mesh=_mesh(),
        scratch_types=[
            pltpu.VMEM((2 * NB, 2, C), jnp.int32),
            pltpu.VMEM((NB, C, D), jnp.float32),
            pltpu.VMEM_SHARED((NP, D), jnp.float32),
        ] + [pltpu.SemaphoreType.DMA] * (3 * NB + 1),
    )
    def k(hn_hbm, idx_hbm, zeros_hbm, out_hbm,
          idx_v, rows_v, acc_sh, *sems):
        c = lax.axis_index("c")
        s = lax.axis_index("s")
        wid = c * NS + s
        sg = sems[:NB]
        si = sems[NB:]
        sz = sems[-1]
        zdesc = pltpu.make_async_copy(
            zeros_hbm, acc_sh.at[pl.ds(s * RPS, RPS)], sz)
        zdesc.start()

        def start_idx(q, b):
            pltpu.async_copy(idx_hbm.at[0, wid, q], idx_v.at[b, 0], si[b])
            pltpu.async_copy(idx_hbm.at[1, wid, q], idx_v.at[b, 1], si[b])

        def wait_idx(q, b):
            pltpu.make_async_copy(
                idx_hbm.at[0, wid, q], idx_v.at[b, 0], si[b]).wait()
            pltpu.make_async_copy(
                idx_hbm.at[1, wid, q], idx_v.at[b, 1], si[b]).wait()

        def start_gather(b):
            pltpu.async_copy(hn_hbm.at[idx_v.at[b, 0]], rows_v.at[b], sg[b])

        def wait_gather(b):
            pltpu.make_async_copy(
                hn_hbm.at[idx_v.at[b, 0]], rows_v.at[b], sg[b]).wait()

        def start_scatter(b):
            pltpu.async_copy(rows_v.at[b], acc_sh.at[idx_v.at[b, 1]], sg[b],
                             add=True)

        def wait_scatter(b):
            pltpu.make_async_copy(
                rows_v.at[b], acc_sh.at[idx_v.at[b, 1]], sg[b]).wait()

        def idx_round(qs, par):
            for b in range(NB):
                start_idx(qs + b, NB * par + b)

        def wait_idx_round(qs, par):
            for b in range(NB):
                wait_idx(qs + b, NB * par + b)

        def gather_round(par):
            for b in range(NB):
                pltpu.async_copy(hn_hbm.at[idx_v.at[NB * par + b, 0]],
                                 rows_v.at[b], sg[b])

        def wait_gather_round(par):
            for b in range(NB):
                pltpu.make_async_copy(hn_hbm.at[idx_v.at[NB * par + b, 0]],
                                      rows_v.at[b], sg[b]).wait()

        def scatter_round(par):
            for b in range(NB):
                pltpu.make_async_copy(hn_hbm.at[idx_v.at[NB * par + b, 0]],
                                      rows_v.at[b], sg[b]).wait()
                pltpu.async_copy(rows_v.at[b],
                                 acc_sh.at[idx_v.at[NB * par + b, 1]], sg[b],
                                 add=True)

        def drain_then_gather(par_s, par_g):
            for b in range(NB):
                pltpu.make_async_copy(rows_v.at[b],
                                      acc_sh.at[idx_v.at[NB * par_s + b, 1]],
                                      sg[b]).wait()
                pltpu.async_copy(hn_hbm.at[idx_v.at[NB * par_g + b, 0]],
                                 rows_v.at[b], sg[b])

        def drain_round(par):
            for b in range(NB):
                pltpu.make_async_copy(rows_v.at[b],
                                      acc_sh.at[idx_v.at[NB * par + b, 1]],
                                      sg[b]).wait()

        idx_round(0, 0)
        wait_idx_round(0, 0)
        gather_round(0)
        idx_round(NB, 1)
        zdesc.wait()
        plsc.subcore_barrier()

        @pl.loop(0, CH - 2 * NB, step=2 * NB)
        def _(j):
            scatter_round(0)
            wait_idx_round(j + NB, 1)
            drain_then_gather(0, 1)
            idx_round(j + 2 * NB, 0)
            scatter_round(1)
            wait_idx_round(j + 2 * NB, 0)
            drain_then_gather(1, 0)
            idx_round(j + 3 * NB, 1)

        scatter_round(0)
        wait_idx_round(CH - NB, 1)
        drain_then_gather(0, 1)
        scatter_round(1)
        drain_round(1)

        plsc.subcore_barrier()
        pltpu.sync_copy(
            acc_sh.at[pl.ds(s * RPS, RPS)],
            out_hbm.at[c, pl.ds(s * RPS, RPS)],
        )

    return k(hn, idx3, zeros128)


_BR = 2000


def _tc_mm1(x, W1):
    def body(x_ref, w_ref, o_ref):
        o_ref[...] = jnp.dot(x_ref[...], w_ref[...],
                             preferred_element_type=jnp.float32)

    return pl.pallas_call(
        body,
        grid=(N // _BR,),
        in_specs=[
            pl.BlockSpec((_BR, D), lambda i: (i, 0)),
            pl.BlockSpec((D, D), lambda i: (0, 0)),
        ],
        out_specs=pl.BlockSpec((_BR, D), lambda i: (i, 0)),
        out_shape=jax.ShapeDtypeStruct((N, D), jnp.float32),
    )(x, W1)


def _tc_scale(h1, degc):
    def body(h_ref, dg_ref, o_ref):
        dinv = lax.rsqrt(dg_ref[...] + 1.0)
        o_ref[...] = h_ref[...] * dinv

    return pl.pallas_call(
        body,
        grid=(N // _BR,),
        in_specs=[
            pl.BlockSpec((_BR, D), lambda i: (i, 0)),
            pl.BlockSpec((_BR, 1), lambda i: (i, 0)),
        ],
        out_specs=pl.BlockSpec((_BR, D), lambda i: (i, 0)),
        out_shape=jax.ShapeDtypeStruct((N, D), jnp.float32),
    )(h1, degc)


def _tc_mid(acca, accb, hn1, degc, b1r, W2):
    def body(aa_ref, ab_ref, hn_ref, dg_ref, b_ref, w_ref, o_ref):
        dinv = lax.rsqrt(dg_ref[...] + 1.0)
        s = aa_ref[...] + ab_ref[...] + hn_ref[...]
        o1 = jnp.maximum(dinv * s + b_ref[...], 0.0)
        h2 = jnp.dot(o1, w_ref[...], preferred_element_type=jnp.float32)
        o_ref[...] = h2 * dinv

    return pl.pallas_call(
        body,
        grid=(N // _BR,),
        in_specs=[
            pl.BlockSpec((_BR, D), lambda i: (i, 0)),
            pl.BlockSpec((_BR, D), lambda i: (i, 0)),
            pl.BlockSpec((_BR, D), lambda i: (i, 0)),
            pl.BlockSpec((_BR, 1), lambda i: (i, 0)),
            pl.BlockSpec((1, D), lambda i: (0, 0)),
            pl.BlockSpec((D, D), lambda i: (0, 0)),
        ],
        out_specs=pl.BlockSpec((_BR, D), lambda i: (i, 0)),
        out_shape=jax.ShapeDtypeStruct((N, D), jnp.float32),
    )(acca, accb, hn1, degc, b1r, W2)


def _tc_fin(acca, accb, hn2, degc):
    def body(aa_ref, ab_ref, hn_ref, dg_ref, o_ref):
        dinv = lax.rsqrt(dg_ref[...] + 1.0)
        z = dinv * (aa_ref[...] + ab_ref[...] + hn_ref[...])
        m = jnp.max(z, axis=1, keepdims=True)
        lse = jnp.log(jnp.sum(jnp.exp(z - m), axis=1, keepdims=True))
        o_ref[...] = z - m - lse

    return pl.pallas_call(
        body,
        grid=(N // _BR,),
        in_specs=[
            pl.BlockSpec((_BR, D), lambda i: (i, 0)),
            pl.BlockSpec((_BR, D), lambda i: (i, 0)),
            pl.BlockSpec((_BR, D), lambda i: (i, 0)),
            pl.BlockSpec((_BR, 1), lambda i: (i, 0)),
        ],
        out_specs=pl.BlockSpec((_BR, D), lambda i: (i, 0)),
        out_shape=jax.ShapeDtypeStruct((N, D), jnp.float32),
    )(acca, accb, hn2, degc)


def kernel(x, edge_index, W1, b1, W2):
    idx3 = edge_index.reshape(2, NW, CH, C)
    dstp = jnp.pad(edge_index[1].reshape(NS, DPS),
                   ((0, 0), (0, DPAD - DPS)),
                   constant_values=NP - 1).reshape(NS, DVR, 128)
    zeros128 = jnp.zeros((RPS, D), jnp.float32)

    deg = _sc_degree(dstp)
    h1 = _tc_mm1(x, W1)
    degc = deg.reshape(NP, 1)
    hn1 = _tc_scale(h1, degc)
    acc1 = _sc_scatter(hn1, idx3, zeros128)
    hn2 = _tc_mid(acc1[0], acc1[1], hn1, degc, b1.reshape(1, D), W2)
    acc2 = _sc_scatter(hn2, idx3, zeros128)
    return _tc_fin(acc2[0], acc2[1], hn2, degc)


# speedup vs baseline: 1.1818x; 1.1818x over previous
import dataclasses
import functools

import jax
import jax.numpy as jnp
from jax import lax
from jax.experimental import pallas as pl
from jax.experimental.pallas import tpu as pltpu
from jax.experimental.pallas import tpu_sc as plsc

N = 10000
E = 320000
D = 128

NC = 2
NS = 16
NW = NC * NS
EPW = E // NW
C = 50
CH = EPW // C
NP = 10240
RPS = NP // NS

DPS = E // NS
DVR = (DPS + 127) // 128 + (1 if DPS % 128 else 0)
DVR = -(-DPS // 128)
DPAD = DVR * 128


def _mesh():
    return plsc.VectorSubcoreMesh(core_axis_name="c", subcore_axis_name="s",
                                  num_cores=NC, num_subcores=NS)


def _cp():
    cp = pltpu.CompilerParams()
    if "needs_layout_passes" in pltpu.CompilerParams.__dataclass_fields__:
        cp = dataclasses.replace(cp, needs_layout_passes=False)
    return cp


def _sc_degree(dstp):
    @functools.partial(
        pl.kernel,
        out_type=jax.ShapeDtypeStruct((NP,), jnp.float32),
        mesh=_mesh(),
        scratch_types=[
            pltpu.VMEM((DVR, 128), jnp.int32),
            pltpu.VMEM((NP,), jnp.float32),
            pltpu.VMEM((RPS,), jnp.float32),
            pltpu.VMEM((RPS,), jnp.float32),
            pltpu.VMEM_SHARED((NS, NP), jnp.float32),
        ],
        compiler_params=_cp(),
    )
    def k(dst_hbm, out_hbm, idx_v, hist_v, acc_l, tmp_l, sh):
        c = lax.axis_index("c")
        s = lax.axis_index("s")
        pltpu.sync_copy(dst_hbm.at[s], idx_v)

        @pl.loop(0, NP, step=16)
        def _(i):
            hist_v[pl.ds(i, 16)] = jnp.zeros((16,), jnp.float32)

        ones = jnp.ones((16,), jnp.float32)

        @pl.loop(0, DVR)
        def _(i):
            for l in range(8):
                plsc.addupdate_scatter(
                    hist_v, [idx_v[i, pl.ds(l * 16, 16)]], ones)

        pltpu.sync_copy(hist_v, sh.at[s])
        plsc.subcore_barrier()

        pltpu.sync_copy(sh.at[0, pl.ds(s * RPS, RPS)], acc_l)
        for w in range(1, NS):
            pltpu.sync_copy(sh.at[w, pl.ds(s * RPS, RPS)], tmp_l)

            @pl.loop(0, RPS, step=16)
            def _(i):
                acc_l[pl.ds(i, 16)] += tmp_l[pl.ds(i, 16)]

        @pl.when(c == 0)
        def _():
            pltpu.sync_copy(acc_l, out_hbm.at[pl.ds(s * RPS, RPS)])

    return k(dstp)


def _sc_scatter(hn, idx3, zeros128):
    NB = 5

    @functools.partial(
        pl.kernel,
        out_type=jax.ShapeDtypeStruct((NC, NP, D), jnp.float32),
        mesh=_mesh(),
        scratch_types=[
            pltpu.VMEM((2 * NB, 2, C), jnp.int32),
            pltpu.VMEM((NB, C, D), jnp.float32),
            pltpu.VMEM_SHARED((NP, D), jnp.float32),
        ] + [pltpu.SemaphoreType.DMA] * (3 * NB + 1),
    )
    def k(hn_hbm, idx_hbm, zeros_hbm, out_hbm,
          idx_v, rows_v, acc_sh, *sems):
        c = lax.axis_index("c")
        s = lax.axis_index("s")
        wid = c * NS + s
        sg = sems[:NB]
        si = sems[NB:]
        sz = sems[-1]
        zdesc = pltpu.make_async_copy(
            zeros_hbm, acc_sh.at[pl.ds(s * RPS, RPS)], sz)
        zdesc.start()

        def start_idx(q, b):
            pltpu.async_copy(idx_hbm.at[0, wid, q], idx_v.at[b, 0], si[b])
            pltpu.async_copy(idx_hbm.at[1, wid, q], idx_v.at[b, 1], si[b])

        def wait_idx(q, b):
            pltpu.make_async_copy(
                idx_hbm.at[0, wid, q], idx_v.at[b, 0], si[b]).wait()
            pltpu.make_async_copy(
                idx_hbm.at[1, wid, q], idx_v.at[b, 1], si[b]).wait()

        def start_gather(b):
            pltpu.async_copy(hn_hbm.at[idx_v.at[b, 0]], rows_v.at[b], sg[b])

        def wait_gather(b):
            pltpu.make_async_copy(
                hn_hbm.at[idx_v.at[b, 0]], rows_v.at[b], sg[b]).wait()

        def start_scatter(b):
            pltpu.async_copy(rows_v.at[b], acc_sh.at[idx_v.at[b, 1]], sg[b],
                             add=True)

        def wait_scatter(b):
            pltpu.make_async_copy(
                rows_v.at[b], acc_sh.at[idx_v.at[b, 1]], sg[b]).wait()

        def idx_round(qs, par):
            for b in range(NB):
                start_idx(qs + b, NB * par + b)

        def wait_idx_round(qs, par):
            for b in range(NB):
                wait_idx(qs + b, NB * par + b)

        def gather_round(par):
            for b in range(NB):
                pltpu.async_copy(hn_hbm.at[idx_v.at[NB * par + b, 0]],
                                 rows_v.at[b], sg[b])

        def wait_gather_round(par):
            for b in range(NB):
                pltpu.make_async_copy(hn_hbm.at[idx_v.at[NB * par + b, 0]],
                                      rows_v.at[b], sg[b]).wait()

        def scatter_round(par):
            for b in range(NB):
                pltpu.make_async_copy(hn_hbm.at[idx_v.at[NB * par + b, 0]],
                                      rows_v.at[b], sg[b]).wait()
                pltpu.async_copy(rows_v.at[b],
                                 acc_sh.at[idx_v.at[NB * par + b, 1]], sg[b],
                                 add=True)

        def drain_then_gather(par_s, par_g):
            for b in range(NB):
                pltpu.make_async_copy(rows_v.at[b],
                                      acc_sh.at[idx_v.at[NB * par_s + b, 1]],
                                      sg[b]).wait()
                pltpu.async_copy(hn_hbm.at[idx_v.at[NB * par_g + b, 0]],
                                 rows_v.at[b], sg[b])

        def drain_round(par):
            for b in range(NB):
                pltpu.make_async_copy(rows_v.at[b],
                                      acc_sh.at[idx_v.at[NB * par + b, 1]],
                                      sg[b]).wait()

        idx_round(0, 0)
        wait_idx_round(0, 0)
        gather_round(0)
        idx_round(NB, 1)
        zdesc.wait()
        plsc.subcore_barrier()

        @pl.loop(0, CH - 2 * NB, step=2 * NB)
        def _(j):
            scatter_round(0)
            wait_idx_round(j + NB, 1)
            drain_then_gather(0, 1)
            idx_round(j + 2 * NB, 0)
            scatter_round(1)
            wait_idx_round(j + 2 * NB, 0)
            drain_then_gather(1, 0)
            idx_round(j + 3 * NB, 1)

        scatter_round(0)
        wait_idx_round(CH - NB, 1)
        drain_then_gather(0, 1)
        scatter_round(1)
        drain_round(1)

        plsc.subcore_barrier()
        pltpu.sync_copy(
            acc_sh.at[pl.ds(s * RPS, RPS)],
            out_hbm.at[c, pl.ds(s * RPS, RPS)],
        )

    return k(hn, idx3, zeros128)


_BR = 2000


def _tc_mm1(x, W1):
    def body(x_ref, w_ref, o_ref):
        o_ref[...] = jnp.dot(x_ref[...], w_ref[...],
                             preferred_element_type=jnp.float32)

    return pl.pallas_call(
        body,
        grid=(N // _BR,),
        in_specs=[
            pl.BlockSpec((_BR, D), lambda i: (i, 0)),
            pl.BlockSpec((D, D), lambda i: (0, 0)),
        ],
        out_specs=pl.BlockSpec((_BR, D), lambda i: (i, 0)),
        out_shape=jax.ShapeDtypeStruct((N, D), jnp.float32),
    )(x, W1)


def _tc_scale(h1, degc):
    def body(h_ref, dg_ref, o_ref):
        dinv = lax.rsqrt(dg_ref[...] + 1.0)
        o_ref[...] = h_ref[...] * dinv

    return pl.pallas_call(
        body,
        grid=(N // _BR,),
        in_specs=[
            pl.BlockSpec((_BR, D), lambda i: (i, 0)),
            pl.BlockSpec((_BR, 1), lambda i: (i, 0)),
        ],
        out_specs=pl.BlockSpec((_BR, D), lambda i: (i, 0)),
        out_shape=jax.ShapeDtypeStruct((N, D), jnp.float32),
    )(h1, degc)


def _tc_mid(acca, accb, hn1, degc, b1r, W2):
    def body(aa_ref, ab_ref, hn_ref, dg_ref, b_ref, w_ref, o_ref):
        dinv = lax.rsqrt(dg_ref[...] + 1.0)
        s = aa_ref[...] + ab_ref[...] + hn_ref[...]
        o1 = jnp.maximum(dinv * s + b_ref[...], 0.0)
        h2 = jnp.dot(o1, w_ref[...], preferred_element_type=jnp.float32)
        o_ref[...] = h2 * dinv

    return pl.pallas_call(
        body,
        grid=(N // _BR,),
        in_specs=[
            pl.BlockSpec((_BR, D), lambda i: (i, 0)),
            pl.BlockSpec((_BR, D), lambda i: (i, 0)),
            pl.BlockSpec((_BR, D), lambda i: (i, 0)),
            pl.BlockSpec((_BR, 1), lambda i: (i, 0)),
            pl.BlockSpec((1, D), lambda i: (0, 0)),
            pl.BlockSpec((D, D), lambda i: (0, 0)),
        ],
        out_specs=pl.BlockSpec((_BR, D), lambda i: (i, 0)),
        out_shape=jax.ShapeDtypeStruct((N, D), jnp.float32),
    )(acca, accb, hn1, degc, b1r, W2)


def _tc_fin(acca, accb, hn2, degc):
    def body(aa_ref, ab_ref, hn_ref, dg_ref, o_ref):
        dinv = lax.rsqrt(dg_ref[...] + 1.0)
        z = dinv * (aa_ref[...] + ab_ref[...] + hn_ref[...])
        m = jnp.max(z, axis=1, keepdims=True)
        lse = jnp.log(jnp.sum(jnp.exp(z - m), axis=1, keepdims=True))
        o_ref[...] = z - m - lse

    return pl.pallas_call(
        body,
        grid=(N // _BR,),
        in_specs=[
            pl.BlockSpec((_BR, D), lambda i: (i, 0)),
            pl.BlockSpec((_BR, D), lambda i: (i, 0)),
            pl.BlockSpec((_BR, D), lambda i: (i, 0)),
            pl.BlockSpec((_BR, 1), lambda i: (i, 0)),
        ],
        out_specs=pl.BlockSpec((_BR, D), lambda i: (i, 0)),
        out_shape=jax.ShapeDtypeStruct((N, D), jnp.float32),
    )(acca, accb, hn2, degc)


def kernel(x, edge_index, W1, b1, W2):
    idx3 = edge_index.reshape(2, NW, CH, C)
    dstp = jnp.pad(edge_index[1].reshape(NS, DPS),
                   ((0, 0), (0, DPAD - DPS)),
                   constant_values=NP - 1).reshape(NS, DVR, 128)
    zeros128 = jnp.zeros((RPS, D), jnp.float32)

    deg = _sc_degree(dstp)
    h1 = _tc_mm1(x, W1)
    degc = deg.reshape(NP, 1)
    hn1 = _tc_scale(h1, degc)
    acc1 = _sc_scatter(hn1, idx3, zeros128)
    hn2 = _tc_mid(acc1[0], acc1[1], hn1, degc, b1.reshape(1, D), W2)
    acc2 = _sc_scatter(hn2, idx3, zeros128)
    return _tc_fin(acc2[0], acc2[1], hn2, degc)


# speedup vs baseline: 1.1865x; 1.0039x over previous
import dataclasses
import functools

import jax
import jax.numpy as jnp
from jax import lax
from jax.experimental import pallas as pl
from jax.experimental.pallas import tpu as pltpu
from jax.experimental.pallas import tpu_sc as plsc

N = 10000
E = 320000
D = 128

NC = 2
NS = 16
NW = NC * NS
EPW = E // NW
C = 50
CH = EPW // C
NP = 10240
RPS = NP // NS

DPS = E // NS
DVR = (DPS + 127) // 128 + (1 if DPS % 128 else 0)
DVR = -(-DPS // 128)
DPAD = DVR * 128


def _mesh():
    return plsc.VectorSubcoreMesh(core_axis_name="c", subcore_axis_name="s",
                                  num_cores=NC, num_subcores=NS)


def _cp():
    cp = pltpu.CompilerParams()
    if "needs_layout_passes" in pltpu.CompilerParams.__dataclass_fields__:
        cp = dataclasses.replace(cp, needs_layout_passes=False)
    return cp


def _sc_degree(dstp):
    @functools.partial(
        pl.kernel,
        out_type=jax.ShapeDtypeStruct((NP,), jnp.float32),
        mesh=_mesh(),
        scratch_types=[
            pltpu.VMEM((DVR, 128), jnp.int32),
            pltpu.VMEM((NP,), jnp.float32),
            pltpu.VMEM((RPS,), jnp.float32),
            pltpu.VMEM((RPS,), jnp.float32),
            pltpu.VMEM_SHARED((NS, NP), jnp.float32),
        ],
        compiler_params=_cp(),
    )
    def k(dst_hbm, out_hbm, idx_v, hist_v, acc_l, tmp_l, sh):
        c = lax.axis_index("c")
        s = lax.axis_index("s")
        pltpu.sync_copy(dst_hbm.at[s], idx_v)

        @pl.loop(0, NP, step=16)
        def _(i):
            hist_v[pl.ds(i, 16)] = jnp.zeros((16,), jnp.float32)

        ones = jnp.ones((16,), jnp.float32)

        @pl.loop(0, DVR)
        def _(i):
            for l in range(8):
                plsc.addupdate_scatter(
                    hist_v, [idx_v[i, pl.ds(l * 16, 16)]], ones)

        pltpu.sync_copy(hist_v, sh.at[s])
        plsc.subcore_barrier()

        pltpu.sync_copy(sh.at[0, pl.ds(s * RPS, RPS)], acc_l)
        for w in range(1, NS):
            pltpu.sync_copy(sh.at[w, pl.ds(s * RPS, RPS)], tmp_l)

            @pl.loop(0, RPS, step=16)
            def _(i):
                acc_l[pl.ds(i, 16)] += tmp_l[pl.ds(i, 16)]

        @pl.when(c == 0)
        def _():
            pltpu.sync_copy(acc_l, out_hbm.at[pl.ds(s * RPS, RPS)])

    return k(dstp)


def _sc_scatter(hn, idx3, zeros128):
    NB = 5

    @functools.partial(
        pl.kernel,
        out_type=jax.ShapeDtypeStruct((NC, NP, D), jnp.float32),
        mesh=_mesh(),
        scratch_types=[
            pltpu.VMEM((2 * NB, 2, C), jnp.int32),
            pltpu.VMEM((NB, C, D), jnp.float32),
            pltpu.VMEM_SHARED((NP, D), jnp.float32),
        ] + [pltpu.SemaphoreType.DMA] * (3 * NB + 1),
    )
    def k(hn_hbm, idx_hbm, zeros_hbm, out_hbm,
          idx_v, rows_v, acc_sh, *sems):
        c = lax.axis_index("c")
        s = lax.axis_index("s")
        wid = c * NS + s
        sg = sems[:NB]
        si = sems[NB:]
        sz = sems[-1]
        zdesc = pltpu.make_async_copy(
            zeros_hbm, acc_sh.at[pl.ds(s * RPS, RPS)], sz)
        zdesc.start()

        def start_idx(q, b):
            pltpu.async_copy(idx_hbm.at[0, wid, q], idx_v.at[b, 0], si[b])
            pltpu.async_copy(idx_hbm.at[1, wid, q], idx_v.at[b, 1], si[b])

        def wait_idx(q, b):
            pltpu.make_async_copy(
                idx_hbm.at[0, wid, q], idx_v.at[b, 0], si[b]).wait()
            pltpu.make_async_copy(
                idx_hbm.at[1, wid, q], idx_v.at[b, 1], si[b]).wait()

        def start_gather(b):
            pltpu.async_copy(hn_hbm.at[idx_v.at[b, 0]], rows_v.at[b], sg[b])

        def wait_gather(b):
            pltpu.make_async_copy(
                hn_hbm.at[idx_v.at[b, 0]], rows_v.at[b], sg[b]).wait()

        def start_scatter(b):
            pltpu.async_copy(rows_v.at[b], acc_sh.at[idx_v.at[b, 1]], sg[b],
                             add=True)

        def wait_scatter(b):
            pltpu.make_async_copy(
                rows_v.at[b], acc_sh.at[idx_v.at[b, 1]], sg[b]).wait()

        def idx_round(qs, par):
            for b in range(NB):
                start_idx(qs + b, NB * par + b)

        def wait_idx_round(qs, par):
            for b in range(NB):
                wait_idx(qs + b, NB * par + b)

        def gather_round(par):
            for b in range(NB):
                pltpu.async_copy(hn_hbm.at[idx_v.at[NB * par + b, 0]],
                                 rows_v.at[b], sg[b])

        def wait_gather_round(par):
            for b in range(NB):
                pltpu.make_async_copy(hn_hbm.at[idx_v.at[NB * par + b, 0]],
                                      rows_v.at[b], sg[b]).wait()

        def scatter_round(par):
            for b in range(NB):
                pltpu.make_async_copy(hn_hbm.at[idx_v.at[NB * par + b, 0]],
                                      rows_v.at[b], sg[b]).wait()
                pltpu.async_copy(rows_v.at[b],
                                 acc_sh.at[idx_v.at[NB * par + b, 1]], sg[b],
                                 add=True)

        def drain_then_gather(par_s, par_g):
            for b in range(NB):
                pltpu.make_async_copy(rows_v.at[b],
                                      acc_sh.at[idx_v.at[NB * par_s + b, 1]],
                                      sg[b]).wait()
                pltpu.async_copy(hn_hbm.at[idx_v.at[NB * par_g + b, 0]],
                                 rows_v.at[b], sg[b])

        def drain_round(par):
            for b in range(NB):
                pltpu.make_async_copy(rows_v.at[b],
                                      acc_sh.at[idx_v.at[NB * par + b, 1]],
                                      sg[b]).wait()

        idx_round(0, 0)
        wait_idx_round(0, 0)
        gather_round(0)
        idx_round(NB, 1)
        zdesc.wait()
        plsc.subcore_barrier()

        @pl.loop(0, CH - 2 * NB, step=2 * NB)
        def _(j):
            scatter_round(0)
            wait_idx_round(j + NB, 1)
            drain_then_gather(0, 1)
            idx_round(j + 2 * NB, 0)
            scatter_round(1)
            wait_idx_round(j + 2 * NB, 0)
            drain_then_gather(1, 0)
            idx_round(j + 3 * NB, 1)

        scatter_round(0)
        wait_idx_round(CH - NB, 1)
        drain_then_gather(0, 1)
        scatter_round(1)
        drain_round(1)

        plsc.subcore_barrier()
        pltpu.sync_copy(
            acc_sh.at[pl.ds(s * RPS, RPS)],
            out_hbm.at[c, pl.ds(s * RPS, RPS)],
        )

    return k(hn, idx3, zeros128)


_BR = 2000


def _tc_mm1s(x, W1, degc):
    def body(x_ref, w_ref, dg_ref, o_ref):
        h1 = jnp.dot(x_ref[...], w_ref[...],
                     preferred_element_type=jnp.float32)
        o_ref[...] = h1 * lax.rsqrt(dg_ref[...] + 1.0)

    return pl.pallas_call(
        body,
        grid=(N // _BR,),
        in_specs=[
            pl.BlockSpec((_BR, D), lambda i: (i, 0)),
            pl.BlockSpec((D, D), lambda i: (0, 0)),
            pl.BlockSpec((_BR, 1), lambda i: (i, 0)),
        ],
        out_specs=pl.BlockSpec((_BR, D), lambda i: (i, 0)),
        out_shape=jax.ShapeDtypeStruct((N, D), jnp.float32),
    )(x, W1, degc)


def _tc_mid(acca, accb, hn1, degc, b1r, W2):
    def body(aa_ref, ab_ref, hn_ref, dg_ref, b_ref, w_ref, o_ref):
        dinv = lax.rsqrt(dg_ref[...] + 1.0)
        s = aa_ref[...] + ab_ref[...] + hn_ref[...]
        o1 = jnp.maximum(dinv * s + b_ref[...], 0.0)
        h2 = jnp.dot(o1, w_ref[...], preferred_element_type=jnp.float32)
        o_ref[...] = h2 * dinv

    return pl.pallas_call(
        body,
        grid=(N // _BR,),
        in_specs=[
            pl.BlockSpec((_BR, D), lambda i: (i, 0)),
            pl.BlockSpec((_BR, D), lambda i: (i, 0)),
            pl.BlockSpec((_BR, D), lambda i: (i, 0)),
            pl.BlockSpec((_BR, 1), lambda i: (i, 0)),
            pl.BlockSpec((1, D), lambda i: (0, 0)),
            pl.BlockSpec((D, D), lambda i: (0, 0)),
        ],
        out_specs=pl.BlockSpec((_BR, D), lambda i: (i, 0)),
        out_shape=jax.ShapeDtypeStruct((N, D), jnp.float32),
    )(acca, accb, hn1, degc, b1r, W2)


def _tc_fin(acca, accb, hn2, degc):
    def body(aa_ref, ab_ref, hn_ref, dg_ref, o_ref):
        dinv = lax.rsqrt(dg_ref[...] + 1.0)
        z = dinv * (aa_ref[...] + ab_ref[...] + hn_ref[...])
        m = jnp.max(z, axis=1, keepdims=True)
        lse = jnp.log(jnp.sum(jnp.exp(z - m), axis=1, keepdims=True))
        o_ref[...] = z - m - lse

    return pl.pallas_call(
        body,
        grid=(N // _BR,),
        in_specs=[
            pl.BlockSpec((_BR, D), lambda i: (i, 0)),
            pl.BlockSpec((_BR, D), lambda i: (i, 0)),
            pl.BlockSpec((_BR, D), lambda i: (i, 0)),
            pl.BlockSpec((_BR, 1), lambda i: (i, 0)),
        ],
        out_specs=pl.BlockSpec((_BR, D), lambda i: (i, 0)),
        out_shape=jax.ShapeDtypeStruct((N, D), jnp.float32),
    )(acca, accb, hn2, degc)


def kernel(x, edge_index, W1, b1, W2):
    idx3 = edge_index.reshape(2, NW, CH, C)
    dstp = jnp.pad(edge_index[1].reshape(NS, DPS),
                   ((0, 0), (0, DPAD - DPS)),
                   constant_values=NP - 1).reshape(NS, DVR, 128)
    zeros128 = jnp.zeros((RPS, D), jnp.float32)

    deg = _sc_degree(dstp)
    degc = deg.reshape(NP, 1)
    hn1 = _tc_mm1s(x, W1, degc)
    acc1 = _sc_scatter(hn1, idx3, zeros128)
    hn2 = _tc_mid(acc1[0], acc1[1], hn1, degc, b1.reshape(1, D), W2)
    acc2 = _sc_scatter(hn2, idx3, zeros128)
    return _tc_fin(acc2[0], acc2[1], hn2, degc)
